# baseline (speedup 1.0000x reference)
import functools

import jax
import jax.numpy as jnp
from jax import lax
from jax.experimental import pallas as pl
from jax.experimental.pallas import tpu as pltpu
from jax.experimental.pallas import tpu_sc as plsc

B, L, V, D = 4096, 200, 1002, 128
NC, NS = 2, 16
NW = NC * NS
TOTAL = B * L
PER_W = TOTAL // NW
CHUNK = 128
K = PER_W // CHUNK
GPB = 2
SUPER = GPB * CHUNK
G = K // GPB
NBUF = 2


def _make_kernel():
    mesh = plsc.VectorSubcoreMesh(core_axis_name="c", subcore_axis_name="s")

    @functools.partial(
        pl.kernel,
        mesh=mesh,
        out_type=jax.ShapeDtypeStruct((NW, PER_W, D), jnp.float32),
        scratch_types=[
            pltpu.VMEM((K, CHUNK), jnp.int32),
            pltpu.VMEM((NBUF, SUPER, D), jnp.float32),
            pltpu.VMEM_SHARED((V, D), jnp.float32),
            pltpu.SemaphoreType.DMA((NBUF,)),
            pltpu.SemaphoreType.DMA((NBUF,)),
        ],
    )
    def emb_lookup(table_hbm, idx_hbm, out_hbm, idx_v, rows_v, table_sp,
                   gsem, osem):
        sid = lax.axis_index("s")
        wid = sid * NC + lax.axis_index("c")
        my_out = out_hbm.at[wid]

        @pl.when(sid == 0)
        def _():
            pltpu.sync_copy(table_hbm, table_sp)

        pltpu.sync_copy(idx_hbm.at[wid], idx_v)
        plsc.subcore_barrier()

        def start_gathers(g, bb):
            for u in range(GPB):
                pltpu.async_copy(table_sp.at[idx_v.at[g * GPB + u]],
                                 rows_v.at[bb].at[pl.ds(u * CHUNK, CHUNK)],
                                 gsem.at[bb])

        def wait_gathers(bb):
            pltpu.make_async_copy(my_out.at[pl.ds(0, SUPER)], rows_v.at[bb],
                                  gsem.at[bb]).wait()

        def start_out(g, bb):
            pltpu.async_copy(rows_v.at[bb], my_out.at[pl.ds(g * SUPER, SUPER)],
                             osem.at[bb])

        def wait_out(bb):
            pltpu.make_async_copy(rows_v.at[bb], my_out.at[pl.ds(0, SUPER)],
                                  osem.at[bb]).wait()

        start_gathers(0, 0)

        def outer(i, carry):
            g0 = i * NBUF
            for bb in range(NBUF):
                g = g0 + bb
                bn = (bb + 1) % NBUF

                @pl.when(g + 1 < G)
                def _():
                    @pl.when(g + 1 >= NBUF)
                    def _():
                        wait_out(bn)
                    start_gathers(g + 1, bn)

                wait_gathers(bb)
                start_out(g, bb)
            return carry

        lax.fori_loop(0, G // NBUF, outer, 0)
        for bb in range(NBUF):
            wait_out(bb)

    return emb_lookup


_emb_lookup = _make_kernel()


@jax.jit
def kernel(seqs, att_mask, word_embedding):
    del att_mask
    idx = seqs.reshape(NW, K, CHUNK)
    out = _emb_lookup(word_embedding, idx)
    return out.reshape(B, L, D)


# speedup vs baseline: 1.0162x; 1.0162x over previous
import functools

import jax
import jax.numpy as jnp
from jax import lax
from jax.experimental import pallas as pl
from jax.experimental.pallas import tpu as pltpu
from jax.experimental.pallas import tpu_sc as plsc

B, L, V, D = 4096, 200, 1002, 128
NC, NS = 2, 16
NW = NC * NS
TOTAL = B * L
PER_W = TOTAL // NW
CHUNK = 128
K = PER_W // CHUNK
NBUF = 4
LOOK = 2


def _make_kernel():
    mesh = plsc.VectorSubcoreMesh(core_axis_name="c", subcore_axis_name="s")

    @functools.partial(
        pl.kernel,
        mesh=mesh,
        out_type=jax.ShapeDtypeStruct((NW, PER_W, D), jnp.float32),
        scratch_types=[
            pltpu.VMEM((K, CHUNK), jnp.int32),
            pltpu.VMEM((NBUF, CHUNK, D), jnp.float32),
            pltpu.VMEM_SHARED((V, D), jnp.float32),
            pltpu.SemaphoreType.DMA((NBUF,)),
            pltpu.SemaphoreType.DMA((NBUF,)),
        ],
    )
    def emb_lookup(table_hbm, idx_hbm, out_hbm, idx_v, rows_v, table_sp,
                   gsem, osem):
        sid = lax.axis_index("s")
        wid = sid * NC + lax.axis_index("c")
        my_out = out_hbm.at[wid]

        @pl.when(sid == 0)
        def _():
            pltpu.sync_copy(table_hbm, table_sp)

        pltpu.sync_copy(idx_hbm.at[wid], idx_v)
        plsc.subcore_barrier()

        def start_gather(j, b):
            pltpu.async_copy(table_sp.at[idx_v.at[j]], rows_v.at[b],
                             gsem.at[b])

        def wait_gather(b):
            pltpu.make_async_copy(table_sp.at[idx_v.at[0]], rows_v.at[b],
                                  gsem.at[b]).wait()

        def start_out(j, b):
            pltpu.async_copy(rows_v.at[b], my_out.at[pl.ds(j * CHUNK, CHUNK)],
                             osem.at[b])

        def wait_out(b):
            pltpu.make_async_copy(rows_v.at[b], my_out.at[pl.ds(0, CHUNK)],
                                  osem.at[b]).wait()

        for j in range(LOOK):
            start_gather(j, j)

        def outer(i, carry):
            j0 = i * NBUF
            for b in range(NBUF):
                j = j0 + b
                jn = j + LOOK
                bn = (b + LOOK) % NBUF

                @pl.when(jn < K)
                def _():
                    @pl.when(jn >= NBUF)
                    def _():
                        wait_out(bn)
                    start_gather(jn, bn)

                wait_gather(b)
                start_out(j, b)
            return carry

        lax.fori_loop(0, K // NBUF, outer, 0)
        for b in range(NBUF):
            wait_out(b)

    return emb_lookup


_emb_lookup = _make_kernel()


@jax.jit
def kernel(seqs, att_mask, word_embedding):
    del att_mask
    idx = seqs.reshape(NW, K, CHUNK)
    out = _emb_lookup(word_embedding, idx)
    return out.reshape(B, L, D)


# speedup vs baseline: 1.0178x; 1.0016x over previous
import functools

import jax
import jax.numpy as jnp
from jax import lax
from jax.experimental import pallas as pl
from jax.experimental.pallas import tpu as pltpu
from jax.experimental.pallas import tpu_sc as plsc

B, L, V, D = 4096, 200, 1002, 128
NC, NS = 2, 16
NW = NC * NS
TOTAL = B * L
PER_W = TOTAL // NW
CHUNK = 128
K = PER_W // CHUNK
NBUF = 4
LOOK = 2


def _make_kernel():
    mesh = plsc.VectorSubcoreMesh(core_axis_name="c", subcore_axis_name="s")

    @functools.partial(
        pl.kernel,
        mesh=mesh,
        out_type=jax.ShapeDtypeStruct((NW, PER_W, D), jnp.float32),
        scratch_types=[
            pltpu.VMEM((K, CHUNK), jnp.int32),
            pltpu.VMEM((NBUF, CHUNK, D), jnp.float32),
            pltpu.VMEM_SHARED((V, D), jnp.float32),
            pltpu.SemaphoreType.DMA((NBUF,)),
            pltpu.SemaphoreType.DMA((NBUF,)),
        ],
    )
    def emb_lookup(table_hbm, idx_hbm, out_hbm, idx_v, rows_v, table_sp,
                   gsem, osem):
        sid = lax.axis_index("s")
        wid = sid * NC + lax.axis_index("c")
        my_out = out_hbm.at[wid]

        @pl.when(sid < NS - 1)
        def _():
            stg = pl.multiple_of(sid * 64, 8)
            pltpu.sync_copy(table_hbm.at[pl.ds(stg, 64)],
                            table_sp.at[pl.ds(stg, 64)])

        @pl.when(sid == NS - 1)
        def _():
            pltpu.sync_copy(table_hbm.at[pl.ds(944, V - 944)],
                            table_sp.at[pl.ds(944, V - 944)])

        pltpu.sync_copy(idx_hbm.at[wid], idx_v)
        plsc.subcore_barrier()

        def start_gather(j, b):
            pltpu.async_copy(table_sp.at[idx_v.at[j]], rows_v.at[b],
                             gsem.at[b])

        def wait_gather(b):
            pltpu.make_async_copy(table_sp.at[idx_v.at[0]], rows_v.at[b],
                                  gsem.at[b]).wait()

        def start_out(j, b):
            pltpu.async_copy(rows_v.at[b], my_out.at[pl.ds(j * CHUNK, CHUNK)],
                             osem.at[b])

        def wait_out(b):
            pltpu.make_async_copy(rows_v.at[b], my_out.at[pl.ds(0, CHUNK)],
                                  osem.at[b]).wait()

        for j in range(LOOK):
            start_gather(j, j)

        def outer(i, carry):
            j0 = i * NBUF
            for b in range(NBUF):
                j = j0 + b
                jn = j + LOOK
                bn = (b + LOOK) % NBUF

                @pl.when(jn < K)
                def _():
                    @pl.when(jn >= NBUF)
                    def _():
                        wait_out(bn)
                    start_gather(jn, bn)

                wait_gather(b)
                start_out(j, b)
            return carry

        lax.fori_loop(0, K // NBUF, outer, 0)
        for b in range(NBUF):
            wait_out(b)

    return emb_lookup


_emb_lookup = _make_kernel()


@jax.jit
def kernel(seqs, att_mask, word_embedding):
    del att_mask
    idx = seqs.reshape(NW, K, CHUNK)
    out = _emb_lookup(word_embedding, idx)
    return out.reshape(B, L, D)
